# Initial kernel scaffold; baseline (speedup 1.0000x reference)
import jax
import jax.numpy as jnp
from jax.experimental import pallas as pl


def kernel(nodes, feat, edge_index, mask, label, W1, att_src1, att_dst1, b1, W2, att_src2, att_dst2, b2):
    raise NotImplementedError("write your pallas kernel here")


# speedup vs baseline: 39.6994x; 39.6994x over previous
import functools

import jax
import jax.numpy as jnp
from jax import lax
from jax.experimental import pallas as pl
from jax.experimental.pallas import tpu as pltpu
from jax.experimental.pallas import tpu_sc as plsc

_N = 10000
_E = 320000
_D = 128
_C = 32
_H1 = 8
_OC1 = 8
_F1 = _H1 * _OC1
_RA1 = 80
_RA2 = 48
_SB = 80
_NTILES = 16
_NCORES = 2
_NW = _NTILES * _NCORES
_ROWS_PER_W = _E // _SB // _NW
_NODES_PER_TILE = _N // _NTILES


def _make_edge_pass(hd, oc, ra, stage_adt):
    mesh = plsc.VectorSubcoreMesh(core_axis_name="c", subcore_axis_name="s")

    def body(src_hbm, dst_hbm, a_hbm, adt_hbm, zero_hbm, out_hbm,
             adt_v, sidx, didx, arows, acc, sem, sem2):
        cid = lax.axis_index("c")
        sid = lax.axis_index("s")
        wid = sid * _NCORES + cid
        if stage_adt:
            pltpu.sync_copy(adt_hbm, adt_v)
        pltpu.sync_copy(
            zero_hbm.at[pl.ds(sid * _NODES_PER_TILE, _NODES_PER_TILE)],
            acc.at[pl.ds(sid * _NODES_PER_TILE, _NODES_PER_TILE)])
        plsc.subcore_barrier()

        row0 = wid * _ROWS_PER_W

        def step(r, carry):
            pltpu.sync_copy(src_hbm.at[row0 + r], sidx)
            pltpu.sync_copy(dst_hbm.at[row0 + r], didx)
            gat = pltpu.async_copy(a_hbm.at[sidx.at[0]], arows, sem)
            if not stage_adt:
                pltpu.async_copy(adt_hbm.at[didx.at[0]], adt_v, sem2).wait()
            gat.wait()
            for o in range(0, _SB, 16):
                lanes = lax.iota(jnp.int32, 16) + o
                dstg = didx[0, pl.ds(o, 16)]
                for k in range(hd):
                    kcol = jnp.full((16,), k, jnp.int32)
                    a_s = plsc.load_gather(arows, [lanes, kcol])
                    if stage_adt:
                        a_d = plsc.load_gather(adt_v, [dstg, kcol])
                    else:
                        a_d = plsc.load_gather(adt_v, [lanes, kcol])
                    al = a_s + a_d
                    al = jnp.where(al > 0.0, al, 0.2 * al)
                    ex = jnp.exp(al)
                    plsc.store_scatter(arows, [lanes, kcol], ex)
                    for j in range(oc):
                        ccol = jnp.full((16,), hd + k * oc + j, jnp.int32)
                        hv = plsc.load_gather(arows, [lanes, ccol])
                        plsc.store_scatter(arows, [lanes, ccol], ex * hv)
            pltpu.sync_copy(arows, acc.at[didx.at[0]], add=True)
            return carry

        lax.fori_loop(0, _ROWS_PER_W, step, 0)
        plsc.subcore_barrier()
        pltpu.sync_copy(
            acc.at[pl.ds(sid * _NODES_PER_TILE, _NODES_PER_TILE)],
            out_hbm.at[pl.ds(cid * _N + sid * _NODES_PER_TILE,
                             _NODES_PER_TILE)])

    return pl.kernel(
        body,
        out_type=jax.ShapeDtypeStruct((_NCORES * _N, ra), jnp.float32),
        mesh=mesh,
        scratch_types=[
            pltpu.VMEM((_N, hd) if stage_adt else (_SB, hd), jnp.float32),
            pltpu.VMEM((1, _SB), jnp.int32),
            pltpu.VMEM((1, _SB), jnp.int32),
            pltpu.VMEM((_SB, ra), jnp.float32),
            pltpu.VMEM_SHARED((_N, ra), jnp.float32),
            pltpu.SemaphoreType.DMA,
            pltpu.SemaphoreType.DMA,
        ],
        compiler_params=pltpu.CompilerParams(use_tc_tiling_on_sc=False,
                                             needs_layout_passes=False),
    )


def _dense1_body(feat_ref, w1_ref, asrc_ref, adst_ref, rrep_ref,
                 a1_ref, adt_ref, self1_ref):
    h = jnp.dot(feat_ref[...], w1_ref[...], preferred_element_type=jnp.float32)
    as1 = jnp.dot(h, asrc_ref[...], preferred_element_type=jnp.float32)
    ad1 = jnp.dot(h, adst_ref[...], preferred_element_type=jnp.float32)
    al = as1 + ad1
    al = jnp.where(al > 0.0, al, 0.2 * al)
    ex0 = jnp.exp(al)
    ex0r = jnp.dot(ex0, rrep_ref[...], preferred_element_type=jnp.float32)
    zpad = jnp.zeros((_N, _RA1 - 1 - _H1 - _F1 + 1), jnp.float32)
    a1_ref[:, 0:_H1] = as1
    a1_ref[:, _H1:_H1 + _F1] = h
    a1_ref[:, _H1 + _F1:_RA1] = zpad
    adt_ref[...] = ad1
    self1_ref[:, 0:_H1] = ex0
    self1_ref[:, _H1:_H1 + _F1] = ex0r * h
    self1_ref[:, _H1 + _F1:_RA1] = zpad


def _dense2_body(p_ref, self1_ref, b1_ref, w2_ref, asv_ref, adv_ref, rrep_ref,
                 a2_ref, adt2_ref, self2_ref):
    acc = p_ref[0:_N, :] + p_ref[_N:2 * _N, :] + self1_ref[...]
    den = acc[:, 0:_H1]
    num = acc[:, _H1:_H1 + _F1]
    denr = jnp.dot(den, rrep_ref[...], preferred_element_type=jnp.float32)
    out1 = num / (denr + 1e-16) + b1_ref[...]
    x2 = jnp.where(out1 > 0.0, out1, jnp.exp(out1) - 1.0)
    h2 = jnp.dot(x2, w2_ref[...], preferred_element_type=jnp.float32)
    as2 = jnp.sum(h2 * asv_ref[...], axis=1, keepdims=True)
    ad2 = jnp.sum(h2 * adv_ref[...], axis=1, keepdims=True)
    al = as2 + ad2
    al = jnp.where(al > 0.0, al, 0.2 * al)
    ex0 = jnp.exp(al)
    zpad = jnp.zeros((_N, _RA2 - 1 - _C), jnp.float32)
    a2_ref[:, 0:1] = as2
    a2_ref[:, 1:1 + _C] = h2
    a2_ref[:, 1 + _C:_RA2] = zpad
    adt2_ref[...] = ad2
    self2_ref[:, 0:1] = ex0
    self2_ref[:, 1:1 + _C] = ex0 * h2
    self2_ref[:, 1 + _C:_RA2] = zpad


def _final_body(q_ref, self2_ref, b2_ref, maskf_ref, label_ref,
                loss_ref, s_ref, labels_ref):
    acc = q_ref[0:_N, :] + q_ref[_N:2 * _N, :] + self2_ref[...]
    scores = acc[:, 1:1 + _C] / (acc[:, 0:1] + 1e-16) + b2_ref[...]
    mf = maskf_ref[...]
    s = jnp.where(mf > 0.0, scores, 0.0)
    s_ref[...] = s
    lab = jnp.where(mf > 0.0, label_ref[...], 0)
    labels_ref[...] = lab
    mx = jnp.max(s, axis=1, keepdims=True)
    logz = jnp.log(jnp.sum(jnp.exp(s - mx), axis=1, keepdims=True)) + mx
    cls = lax.broadcasted_iota(jnp.int32, (_N, _C), 1)
    sel = jnp.sum(jnp.where(cls == lab, s, 0.0), axis=1, keepdims=True)
    nll = logz - sel
    loss_ref[...] = (jnp.sum(nll * mf, axis=0, keepdims=True) /
                     jnp.sum(mf, axis=0, keepdims=True))


def kernel(nodes, feat, edge_index, mask, label,
           W1, att_src1, att_dst1, b1, W2, att_src2, att_dst2, b2):
    del nodes
    f32 = jnp.float32
    eye = jnp.eye(_H1, dtype=f32)
    asrc = (att_src1.astype(f32)[:, :, None] * eye[:, None, :]).reshape(_F1, _H1)
    adst = (att_dst1.astype(f32)[:, :, None] * eye[:, None, :]).reshape(_F1, _H1)
    rrep = jnp.repeat(eye, _OC1, axis=1)

    a1, adt1, self1 = pl.pallas_call(
        _dense1_body,
        out_shape=[
            jax.ShapeDtypeStruct((_N, _RA1), f32),
            jax.ShapeDtypeStruct((_N, _H1), f32),
            jax.ShapeDtypeStruct((_N, _RA1), f32),
        ],
    )(feat.astype(f32), W1.astype(f32), asrc, adst, rrep)

    src2d = edge_index[0].reshape(_E // _SB, 1, _SB)
    dst2d = edge_index[1].reshape(_E // _SB, 1, _SB)

    p1 = _make_edge_pass(_H1, _OC1, _RA1, False)(
        src2d, dst2d, a1, adt1, jnp.zeros((_N, _RA1), f32))

    a2, adt2, self2 = pl.pallas_call(
        _dense2_body,
        out_shape=[
            jax.ShapeDtypeStruct((_N, _RA2), f32),
            jax.ShapeDtypeStruct((_N, 1), f32),
            jax.ShapeDtypeStruct((_N, _RA2), f32),
        ],
    )(p1, self1, b1.astype(f32).reshape(1, _F1), W2.astype(f32),
      att_src2.astype(f32), att_dst2.astype(f32), rrep)

    q = _make_edge_pass(1, _C, _RA2, True)(
        src2d, dst2d, a2, adt2, jnp.zeros((_N, _RA2), f32))

    loss2d, s, labels2d = pl.pallas_call(
        _final_body,
        out_shape=[
            jax.ShapeDtypeStruct((1, 1), f32),
            jax.ShapeDtypeStruct((_N, _C), f32),
            jax.ShapeDtypeStruct((_N, 1), jnp.int32),
        ],
    )(q, self2, b2.astype(f32).reshape(1, _C),
      mask.astype(f32).reshape(_N, 1), label.astype(jnp.int32).reshape(_N, 1))

    return (loss2d[0, 0], s, labels2d.reshape(_N))


# speedup vs baseline: 57.6621x; 1.4525x over previous
import functools

import jax
import jax.numpy as jnp
from jax import lax
from jax.experimental import pallas as pl
from jax.experimental.pallas import tpu as pltpu
from jax.experimental.pallas import tpu_sc as plsc

_N = 10000
_E = 320000
_D = 128
_C = 32
_H1 = 8
_OC1 = 8
_F1 = _H1 * _OC1
_RA1 = 80
_RA2 = 48
_SB = 80
_NTILES = 16
_NCORES = 2
_NW = _NTILES * _NCORES
_ROWS_PER_W = _E // _SB // _NW
_NODES_PER_TILE = _N // _NTILES


_NBUF = 5
_NWIN = _ROWS_PER_W // _NBUF


def _make_edge_pass(hd, oc, ra, stage_adt):
    mesh = plsc.VectorSubcoreMesh(core_axis_name="c", subcore_axis_name="s")

    def body(*refs):
        (edges_hbm, a_hbm, adt_hbm, zero_hbm, out_hbm,
         adt_v, ibuf, arows, acc) = refs[:9]
        semg = refs[9:9 + _NBUF]
        sems = refs[9 + _NBUF:9 + 2 * _NBUF]
        sema = refs[9 + 2 * _NBUF:]
        cid = lax.axis_index("c")
        sid = lax.axis_index("s")
        wid = sid * _NCORES + cid
        if stage_adt:
            pltpu.sync_copy(adt_hbm, adt_v)
        pltpu.sync_copy(
            zero_hbm.at[pl.ds(sid * _NODES_PER_TILE, _NODES_PER_TILE)],
            acc.at[pl.ds(sid * _NODES_PER_TILE, _NODES_PER_TILE)])
        plsc.subcore_barrier()

        def window(wi, carry):
            gw = wid * _NWIN + wi
            pltpu.sync_copy(edges_hbm.at[gw], ibuf)
            gat = []
            adg = []
            for b in range(_NBUF):
                gat.append(pltpu.async_copy(
                    a_hbm.at[ibuf.at[b, 0]], arows.at[b], semg[b]))
                if not stage_adt:
                    adg.append(pltpu.async_copy(
                        adt_hbm.at[ibuf.at[b, 1]], adt_v.at[b], sema[b]))
            scat = []
            for b in range(_NBUF):
                gat[b].wait()
                if not stage_adt:
                    adg[b].wait()
                ar = arows.at[b]

                def group(g, c, b=b, ar=ar):
                    o16 = pl.multiple_of(g * 16, 16)
                    lanes = lax.iota(jnp.int32, 16) + o16
                    dstg = ibuf[b, 1, pl.ds(o16, 16)]
                    for k in range(hd):
                        kcol = jnp.full((16,), k, jnp.int32)
                        a_s = plsc.load_gather(ar, [lanes, kcol])
                        if stage_adt:
                            a_d = plsc.load_gather(adt_v, [dstg, kcol])
                        else:
                            a_d = plsc.load_gather(adt_v.at[b], [lanes, kcol])
                        al = a_s + a_d
                        al = jnp.where(al > 0.0, al, 0.2 * al)
                        ex = jnp.exp(al)
                        plsc.store_scatter(ar, [lanes, kcol], ex)
                        for j in range(oc):
                            ccol = jnp.full((16,), hd + k * oc + j, jnp.int32)
                            hv = plsc.load_gather(ar, [lanes, ccol])
                            plsc.store_scatter(ar, [lanes, ccol], ex * hv)
                    return c

                lax.fori_loop(0, _SB // 16, group, 0)
                scat.append(pltpu.async_copy(
                    arows.at[b], acc.at[ibuf.at[b, 1]], sems[b], add=True))
            for b in range(_NBUF):
                scat[b].wait()
            return carry

        lax.fori_loop(0, _NWIN, window, 0)
        plsc.subcore_barrier()
        pltpu.sync_copy(
            acc.at[pl.ds(sid * _NODES_PER_TILE, _NODES_PER_TILE)],
            out_hbm.at[pl.ds(cid * _N + sid * _NODES_PER_TILE,
                             _NODES_PER_TILE)])

    scratch = [
        pltpu.VMEM((_N, hd) if stage_adt else (_NBUF, _SB, hd), jnp.float32),
        pltpu.VMEM((_NBUF, 2, _SB), jnp.int32),
        pltpu.VMEM((_NBUF, _SB, ra), jnp.float32),
        pltpu.VMEM_SHARED((_N, ra), jnp.float32),
    ]
    scratch += [pltpu.SemaphoreType.DMA] * (2 * _NBUF)
    if not stage_adt:
        scratch += [pltpu.SemaphoreType.DMA] * _NBUF

    return pl.kernel(
        body,
        out_type=jax.ShapeDtypeStruct((_NCORES * _N, ra), jnp.float32),
        mesh=mesh,
        scratch_types=scratch,
        compiler_params=pltpu.CompilerParams(use_tc_tiling_on_sc=False,
                                             needs_layout_passes=False),
    )


def _dense1_body(feat_ref, w1_ref, asrc_ref, adst_ref, rrep_ref,
                 a1_ref, adt_ref, self1_ref):
    h = jnp.dot(feat_ref[...], w1_ref[...], preferred_element_type=jnp.float32)
    as1 = jnp.dot(h, asrc_ref[...], preferred_element_type=jnp.float32)
    ad1 = jnp.dot(h, adst_ref[...], preferred_element_type=jnp.float32)
    al = as1 + ad1
    al = jnp.where(al > 0.0, al, 0.2 * al)
    ex0 = jnp.exp(al)
    ex0r = jnp.dot(ex0, rrep_ref[...], preferred_element_type=jnp.float32)
    zpad = jnp.zeros((_N, _RA1 - 1 - _H1 - _F1 + 1), jnp.float32)
    a1_ref[:, 0:_H1] = as1
    a1_ref[:, _H1:_H1 + _F1] = h
    a1_ref[:, _H1 + _F1:_RA1] = zpad
    adt_ref[...] = ad1
    self1_ref[:, 0:_H1] = ex0
    self1_ref[:, _H1:_H1 + _F1] = ex0r * h
    self1_ref[:, _H1 + _F1:_RA1] = zpad


def _dense2_body(p_ref, self1_ref, b1_ref, w2_ref, asv_ref, adv_ref, rrep_ref,
                 a2_ref, adt2_ref, self2_ref):
    acc = p_ref[0:_N, :] + p_ref[_N:2 * _N, :] + self1_ref[...]
    den = acc[:, 0:_H1]
    num = acc[:, _H1:_H1 + _F1]
    denr = jnp.dot(den, rrep_ref[...], preferred_element_type=jnp.float32)
    out1 = num / (denr + 1e-16) + b1_ref[...]
    x2 = jnp.where(out1 > 0.0, out1, jnp.exp(out1) - 1.0)
    h2 = jnp.dot(x2, w2_ref[...], preferred_element_type=jnp.float32)
    as2 = jnp.sum(h2 * asv_ref[...], axis=1, keepdims=True)
    ad2 = jnp.sum(h2 * adv_ref[...], axis=1, keepdims=True)
    al = as2 + ad2
    al = jnp.where(al > 0.0, al, 0.2 * al)
    ex0 = jnp.exp(al)
    zpad = jnp.zeros((_N, _RA2 - 1 - _C), jnp.float32)
    a2_ref[:, 0:1] = as2
    a2_ref[:, 1:1 + _C] = h2
    a2_ref[:, 1 + _C:_RA2] = zpad
    adt2_ref[...] = ad2
    self2_ref[:, 0:1] = ex0
    self2_ref[:, 1:1 + _C] = ex0 * h2
    self2_ref[:, 1 + _C:_RA2] = zpad


def _final_body(q_ref, self2_ref, b2_ref, maskf_ref, label_ref,
                loss_ref, s_ref, labels_ref):
    acc = q_ref[0:_N, :] + q_ref[_N:2 * _N, :] + self2_ref[...]
    scores = acc[:, 1:1 + _C] / (acc[:, 0:1] + 1e-16) + b2_ref[...]
    mf = maskf_ref[...]
    s = jnp.where(mf > 0.0, scores, 0.0)
    s_ref[...] = s
    lab = jnp.where(mf > 0.0, label_ref[...], 0)
    labels_ref[...] = lab
    mx = jnp.max(s, axis=1, keepdims=True)
    logz = jnp.log(jnp.sum(jnp.exp(s - mx), axis=1, keepdims=True)) + mx
    cls = lax.broadcasted_iota(jnp.int32, (_N, _C), 1)
    sel = jnp.sum(jnp.where(cls == lab, s, 0.0), axis=1, keepdims=True)
    nll = logz - sel
    loss_ref[...] = (jnp.sum(nll * mf, axis=0, keepdims=True) /
                     jnp.sum(mf, axis=0, keepdims=True))


def kernel(nodes, feat, edge_index, mask, label,
           W1, att_src1, att_dst1, b1, W2, att_src2, att_dst2, b2):
    del nodes
    f32 = jnp.float32
    eye = jnp.eye(_H1, dtype=f32)
    asrc = (att_src1.astype(f32)[:, :, None] * eye[:, None, :]).reshape(_F1, _H1)
    adst = (att_dst1.astype(f32)[:, :, None] * eye[:, None, :]).reshape(_F1, _H1)
    rrep = jnp.repeat(eye, _OC1, axis=1)

    a1, adt1, self1 = pl.pallas_call(
        _dense1_body,
        out_shape=[
            jax.ShapeDtypeStruct((_N, _RA1), f32),
            jax.ShapeDtypeStruct((_N, _H1), f32),
            jax.ShapeDtypeStruct((_N, _RA1), f32),
        ],
    )(feat.astype(f32), W1.astype(f32), asrc, adst, rrep)

    edges = jnp.stack([edge_index[0].reshape(_E // _SB, _SB),
                       edge_index[1].reshape(_E // _SB, _SB)],
                      axis=1).reshape(_E // _SB // _NBUF, _NBUF, 2, _SB)

    p1 = _make_edge_pass(_H1, _OC1, _RA1, False)(
        edges, a1, adt1, jnp.zeros((_N, _RA1), f32))

    a2, adt2, self2 = pl.pallas_call(
        _dense2_body,
        out_shape=[
            jax.ShapeDtypeStruct((_N, _RA2), f32),
            jax.ShapeDtypeStruct((_N, 1), f32),
            jax.ShapeDtypeStruct((_N, _RA2), f32),
        ],
    )(p1, self1, b1.astype(f32).reshape(1, _F1), W2.astype(f32),
      att_src2.astype(f32), att_dst2.astype(f32), rrep)

    q = _make_edge_pass(1, _C, _RA2, True)(
        edges, a2, adt2, jnp.zeros((_N, _RA2), f32))

    loss2d, s, labels2d = pl.pallas_call(
        _final_body,
        out_shape=[
            jax.ShapeDtypeStruct((1, 1), f32),
            jax.ShapeDtypeStruct((_N, _C), f32),
            jax.ShapeDtypeStruct((_N, 1), jnp.int32),
        ],
    )(q, self2, b2.astype(f32).reshape(1, _C),
      mask.astype(f32).reshape(_N, 1), label.astype(jnp.int32).reshape(_N, 1))

    return (loss2d[0, 0], s, labels2d.reshape(_N))
